# baseline (speedup 1.0000x reference)
import functools

import jax
import jax.numpy as jnp
from jax import lax
from jax.experimental import pallas as pl
from jax.experimental.pallas import tpu as pltpu
from jax.experimental.pallas import tpu_sc as plsc

N = 10000
E = 320000
F_IN = 128
H = 64
C = 10

NC = 2
NS = 16
NW = NC * NS
CH = 128
NCHUNK = 81
EPW = NCHUNK * CH
E2 = EPW * NW
PAD = E2 - E - N
NP = 10112
TPT = NP // NS
ZR = 158
NH = 5000
SPR = NP * H // (2 * H)

_MESH = plsc.VectorSubcoreMesh(core_axis_name="c", subcore_axis_name="s")
_SC_PARAMS = pltpu.CompilerParams(use_tc_tiling_on_sc=False,
                                  disable_bounds_checks=True)


@functools.partial(
    pl.kernel,
    out_type=jax.ShapeDtypeStruct((NC * NP, 8), jnp.float32),
    mesh=_MESH,
    compiler_params=_SC_PARAMS,
    scratch_types=[
        pltpu.VMEM((NCHUNK, CH), jnp.int32),
        pltpu.VMEM((CH, 8), jnp.float32),
        pltpu.VMEM((ZR, 8), jnp.float32),
        pltpu.VMEM_SHARED((NP, 8), jnp.float32),
    ],
)
def _sc_degree(col_hbm, ones_hbm, zeros_hbm, out_hbm, cidx, ones_v, zbuf, acc):
    c = lax.axis_index("c")
    s = lax.axis_index("s")
    wid = s * NC + c

    pltpu.sync_copy(col_hbm.at[wid], cidx)
    pltpu.sync_copy(ones_hbm, ones_v)
    pltpu.sync_copy(zeros_hbm, zbuf)

    def zero_body(j, _):
        pltpu.sync_copy(zbuf, acc.at[pl.ds(s * TPT + j * ZR, ZR)])
        return _

    lax.fori_loop(0, TPT // ZR, zero_body, None)
    plsc.subcore_barrier()

    def body(k, _):
        pltpu.sync_copy(ones_v, acc.at[cidx.at[k]], add=True)
        return _

    lax.fori_loop(0, NCHUNK, body, None)
    plsc.subcore_barrier()

    def out_body(j, _):
        r0 = s * TPT + j * ZR
        pltpu.sync_copy(acc.at[pl.ds(r0, ZR)], zbuf)
        pltpu.sync_copy(zbuf, out_hbm.at[pl.ds(c * NP + r0, ZR)])
        return _

    lax.fori_loop(0, TPT // ZR, out_body, None)


@functools.partial(
    pl.kernel,
    out_type=jax.ShapeDtypeStruct((NC * NP, H), jnp.float32),
    mesh=_MESH,
    compiler_params=_SC_PARAMS,
    scratch_types=[
        pltpu.VMEM((NCHUNK, CH), jnp.int32),
        pltpu.VMEM((NCHUNK, CH), jnp.int32),
        pltpu.VMEM((CH, H), jnp.float32),
        pltpu.VMEM((CH, H), jnp.float32),
        pltpu.VMEM((CH, H), jnp.float32),
        pltpu.VMEM((CH, H), jnp.float32),
        pltpu.VMEM((ZR, H), jnp.float32),
        pltpu.VMEM_SHARED((NP, H), jnp.float32),
        pltpu.SemaphoreType.DMA,
        pltpu.SemaphoreType.DMA,
        pltpu.SemaphoreType.DMA,
        pltpu.SemaphoreType.DMA,
    ],
)
def _sc_aggregate(y_hbm, row_hbm, col_hbm, zeros_hbm, out_hbm,
                  ridx, cidx, buf0, buf1, buf2, buf3, zbuf, acc,
                  g0, g1, g2, g3):
    c = lax.axis_index("c")
    s = lax.axis_index("s")
    wid = s * NC + c

    pltpu.sync_copy(row_hbm.at[wid], ridx)
    pltpu.sync_copy(col_hbm.at[wid], cidx)
    pltpu.sync_copy(zeros_hbm, zbuf)

    def zero_body(j, _):
        pltpu.sync_copy(zbuf, acc.at[pl.ds(s * TPT + j * ZR, ZR)])
        return _

    lax.fori_loop(0, TPT // ZR, zero_body, None)
    plsc.subcore_barrier()

    bufs = (buf0, buf1, buf2, buf3)
    sems = (g0, g1, g2, g3)
    nb = len(bufs)
    for b in range(nb):
        pltpu.async_copy(y_hbm.at[ridx.at[b]], bufs[b], sems[b])

    def body(i, _):
        k = nb * i
        for b in range(nb):
            pltpu.make_async_copy(y_hbm.at[ridx.at[k + b]], bufs[b],
                                  sems[b]).wait()
            pltpu.sync_copy(bufs[b], acc.at[cidx.at[k + b]], add=True)

            @pl.when(k + b + nb < NCHUNK)
            def _g():
                pltpu.async_copy(y_hbm.at[ridx.at[k + b + nb]], bufs[b],
                                 sems[b])

        return _

    lax.fori_loop(0, NCHUNK // nb, body, None)
    tail = (NCHUNK // nb) * nb
    for b in range(NCHUNK - tail):
        pltpu.make_async_copy(y_hbm.at[ridx.at[tail + b]], bufs[b],
                              sems[b]).wait()
        pltpu.sync_copy(bufs[b], acc.at[cidx.at[tail + b]], add=True)

    plsc.subcore_barrier()

    def out_body(j, _):
        r0 = s * TPT + j * ZR
        pltpu.sync_copy(acc.at[pl.ds(r0, ZR)], zbuf)
        pltpu.sync_copy(zbuf, out_hbm.at[pl.ds(c * NP + r0, ZR)])
        return _

    lax.fori_loop(0, TPT // ZR, out_body, None)


TAILN = E2 - E


def _tc_edges_body(ei_ref, rowx_ref, colx_ref):
    ei = ei_ref[...]
    j = lax.broadcasted_iota(jnp.int32, (TAILN,), 0)
    rowx_ref[pl.ds(0, E)] = ei[0, :]
    rowx_ref[pl.ds(E, TAILN)] = jnp.where(j < N, j, j - N)
    colx_ref[pl.ds(0, E)] = ei[1, :]
    colx_ref[pl.ds(E, TAILN)] = jnp.where(j < N, j, N + (j - N) % (NP - N))


def _tc_edges(edge_index):
    return pl.pallas_call(
        _tc_edges_body,
        out_shape=[
            jax.ShapeDtypeStruct((E2,), jnp.int32),
            jax.ShapeDtypeStruct((E2,), jnp.int32),
        ],
    )(edge_index)


DGR = NP * 8 // 128


def _tc_dinv_body(deg_ref, p_ref, dinv_ref):
    v = deg_ref[...]
    dall = lax.rsqrt(v[:DGR] + v[DGR:])
    drep = jnp.broadcast_to(dall[:, None, :], (DGR, 8, 128))
    drep = drep.reshape(8 * DGR, 128)
    p = p_ref[...]
    rq = lax.broadcasted_iota(jnp.int32, (8 * DGR, 1), 0) % 8
    out = jnp.zeros((8 * DGR, 128), jnp.float32)
    for q in range(8):
        sel = jnp.dot(drep, p[128 * q:128 * (q + 1)],
                      preferred_element_type=jnp.float32,
                      precision=lax.Precision.HIGHEST)
        out = jnp.where(rq == q, sel, out)
    dinv_ref[...] = out[:NH]


def _tc_dinv(deg_parts, perms):
    return pl.pallas_call(
        _tc_dinv_body,
        out_shape=jax.ShapeDtypeStruct((NH, 2 * H), jnp.float32),
    )(deg_parts.reshape(2 * DGR, 128), perms)


def _dinv_perms():
    l64 = (jnp.arange(128) >= 64).astype(jnp.int32)
    rows = []
    for q in range(8):
        src = 16 * q + 8 * l64
        rows.append((jnp.arange(128)[:, None] == src[None, :])
                    .astype(jnp.float32))
    return jnp.concatenate(rows, axis=0)


def _tc_y0_body(x2_ref, w0_ref, dinv_ref, y_ref):
    xw = jnp.dot(x2_ref[...], w0_ref[...], preferred_element_type=jnp.float32)
    y_ref[...] = xw * dinv_ref[...]


def _tc_y0(x2, W0blk, dinv128):
    return pl.pallas_call(
        _tc_y0_body,
        out_shape=jax.ShapeDtypeStruct((NH, 2 * H), jnp.float32),
    )(x2, W0blk, dinv128)


def _bn_relu(o, g2, be2):
    mu = jnp.mean(o, axis=0, keepdims=True)
    sq = jnp.mean(o * o, axis=0, keepdims=True)
    mu64 = (mu[:, :H] + mu[:, H:]) * 0.5
    sq64 = (sq[:, :H] + sq[:, H:]) * 0.5
    var64 = sq64 - mu64 * mu64
    mu2 = jnp.concatenate([mu64, mu64], axis=1)
    rstd2 = jnp.concatenate([lax.rsqrt(var64 + 1e-5),
                             lax.rsqrt(var64 + 1e-5)], axis=1)
    return jnp.maximum((o - mu2) * rstd2 * g2 + be2, 0.0)


def _tc_post_body(sp_ref, dinv_ref, b_ref, g_ref, be_ref, wn_ref, yn_ref):
    dinv = dinv_ref[...]
    sp = sp_ref[...]
    o = dinv * (sp[:NH] + sp[SPR:SPR + NH]) + b_ref[...]
    h = _bn_relu(o, g_ref[...], be_ref[...])
    yn_ref[...] = jnp.dot(h, wn_ref[...],
                          preferred_element_type=jnp.float32) * dinv


def _tc_post(sp128, dinv128, b2, g2, be2, Wnblk):
    return pl.pallas_call(
        _tc_post_body,
        out_shape=jax.ShapeDtypeStruct((NH, 2 * H), jnp.float32),
    )(sp128, dinv128, b2, g2, be2, Wnblk)


def _tc_final_body(sp_ref, dinv_ref, b_ref, g_ref, be_ref,
                   wc1_ref, bc1_ref, wc2_ref, bc2_ref, out_ref):
    sp = sp_ref[...]
    o = dinv_ref[...] * (sp[:NH] + sp[SPR:SPR + NH]) + b_ref[...]
    h = _bn_relu(o, g_ref[...], be_ref[...])
    hc = jnp.maximum(
        jnp.dot(h, wc1_ref[...], preferred_element_type=jnp.float32)
        + bc1_ref[...], 0.0)
    out_ref[...] = (
        jnp.dot(hc, wc2_ref[...], preferred_element_type=jnp.float32)
        + bc2_ref[...])


def _tc_final(sp128, dinv128, b2, g2, be2, Wc1blk, bc1_2, Wc2blk, bc2_2):
    return pl.pallas_call(
        _tc_final_body,
        out_shape=jax.ShapeDtypeStruct((NH, 2 * C), jnp.float32),
    )(sp128, dinv128, b2, g2, be2, Wc1blk, bc1_2, Wc2blk, bc2_2)


def _blockdiag(W):
    fi, fo = W.shape
    Z = jnp.zeros((fi, fo), W.dtype)
    return jnp.concatenate(
        [jnp.concatenate([W, Z], axis=1), jnp.concatenate([Z, W], axis=1)],
        axis=0)


def _pair(v):
    return jnp.concatenate([v, v]).reshape(1, 2 * v.shape[0])


def kernel(x, edge_index, W0, b0, W1, b1, W2, b2, g0, be0, g1, be1, g2, be2,
           Wc1, bc1, Wc2, bc2):
    rowx, colx = _tc_edges(edge_index)
    row3 = rowx.reshape(NW, NCHUNK, CH)
    col3 = colx.reshape(NW, NCHUNK, CH)
    ones8 = jnp.ones((CH, 8), jnp.float32)
    zeros8 = jnp.zeros((ZR, 8), jnp.float32)
    zerosH = jnp.zeros((ZR, H), jnp.float32)

    x2 = x.reshape(NH, 2 * F_IN)

    deg_parts = _sc_degree(col3, ones8, zeros8)
    dinv128 = _tc_dinv(deg_parts, _dinv_perms())
    y0 = _tc_y0(x2, _blockdiag(W0), dinv128)

    s0 = _sc_aggregate(y0.reshape(2 * NH, H), row3, col3,
                       zerosH).reshape(NC * NP * H // (2 * H), 2 * H)
    y1 = _tc_post(s0, dinv128, _pair(b0), _pair(g0), _pair(be0),
                  _blockdiag(W1))
    s1 = _sc_aggregate(y1.reshape(2 * NH, H), row3, col3,
                       zerosH).reshape(NC * NP * H // (2 * H), 2 * H)
    y2 = _tc_post(s1, dinv128, _pair(b1), _pair(g1), _pair(be1),
                  _blockdiag(W2))
    s2 = _sc_aggregate(y2.reshape(2 * NH, H), row3, col3,
                       zerosH).reshape(NC * NP * H // (2 * H), 2 * H)
    out2 = _tc_final(s2, dinv128, _pair(b2), _pair(g2), _pair(be2),
                     _blockdiag(Wc1), _pair(bc1), _blockdiag(Wc2), _pair(bc2))
    return out2.reshape(N, C)


# speedup vs baseline: 1.0495x; 1.0495x over previous
import functools

import jax
import jax.numpy as jnp
from jax import lax
from jax.experimental import pallas as pl
from jax.experimental.pallas import tpu as pltpu
from jax.experimental.pallas import tpu_sc as plsc

N = 10000
E = 320000
F_IN = 128
H = 64
C = 10

NC = 2
NS = 16
NW = NC * NS
CH = 128
NCHUNK = 81
EPW = NCHUNK * CH
E2 = EPW * NW
PAD = E2 - E - N
NP = 10112
TPT = NP // NS
ZR = 158
NH = 5000
SPR = NP * H // (2 * H)

_MESH = plsc.VectorSubcoreMesh(core_axis_name="c", subcore_axis_name="s")
_SC_PARAMS = pltpu.CompilerParams(use_tc_tiling_on_sc=False,
                                  disable_bounds_checks=True)


@functools.partial(
    pl.kernel,
    out_type=jax.ShapeDtypeStruct((NC * NP, 8), jnp.float32),
    mesh=_MESH,
    compiler_params=_SC_PARAMS,
    scratch_types=[
        pltpu.VMEM((NCHUNK, CH), jnp.int32),
        pltpu.VMEM((CH, 8), jnp.float32),
        pltpu.VMEM((ZR, 8), jnp.float32),
        pltpu.VMEM_SHARED((NP, 8), jnp.float32),
    ],
)
def _sc_degree(col_hbm, ones_hbm, zeros_hbm, out_hbm, cidx, ones_v, zbuf, acc):
    c = lax.axis_index("c")
    s = lax.axis_index("s")
    wid = s * NC + c

    pltpu.sync_copy(col_hbm.at[wid], cidx)
    pltpu.sync_copy(ones_hbm, ones_v)
    pltpu.sync_copy(zeros_hbm, zbuf)

    def zero_body(j, _):
        pltpu.sync_copy(zbuf, acc.at[pl.ds(s * TPT + j * ZR, ZR)])
        return _

    lax.fori_loop(0, TPT // ZR, zero_body, None)
    plsc.subcore_barrier()

    def body(k, _):
        pltpu.sync_copy(ones_v, acc.at[cidx.at[k]], add=True)
        return _

    lax.fori_loop(0, NCHUNK, body, None)
    plsc.subcore_barrier()

    def out_body(j, _):
        r0 = s * TPT + j * ZR
        pltpu.sync_copy(acc.at[pl.ds(r0, ZR)], zbuf)
        pltpu.sync_copy(zbuf, out_hbm.at[pl.ds(c * NP + r0, ZR)])
        return _

    lax.fori_loop(0, TPT // ZR, out_body, None)


@functools.partial(
    pl.kernel,
    out_type=jax.ShapeDtypeStruct((NC * NP, H), jnp.float32),
    mesh=_MESH,
    compiler_params=_SC_PARAMS,
    scratch_types=[
        pltpu.VMEM((NCHUNK, CH), jnp.int32),
        pltpu.VMEM((NCHUNK, CH), jnp.int32),
        pltpu.VMEM((CH, H), jnp.float32),
        pltpu.VMEM((CH, H), jnp.float32),
        pltpu.VMEM((CH, H), jnp.float32),
        pltpu.VMEM((CH, H), jnp.float32),
        pltpu.VMEM((ZR, H), jnp.float32),
        pltpu.VMEM_SHARED((NP, H), jnp.float32),
        pltpu.SemaphoreType.DMA,
        pltpu.SemaphoreType.DMA,
        pltpu.SemaphoreType.DMA,
        pltpu.SemaphoreType.DMA,
    ],
)
def _sc_aggregate(y_hbm, row_hbm, col_hbm, zeros_hbm, out_hbm,
                  ridx, cidx, buf0, buf1, buf2, buf3, zbuf, acc,
                  g0, g1, g2, g3):
    c = lax.axis_index("c")
    s = lax.axis_index("s")
    wid = s * NC + c

    pltpu.sync_copy(row_hbm.at[wid], ridx)
    pltpu.sync_copy(col_hbm.at[wid], cidx)
    pltpu.sync_copy(zeros_hbm, zbuf)

    def zero_body(j, _):
        pltpu.sync_copy(zbuf, acc.at[pl.ds(s * TPT + j * ZR, ZR)])
        return _

    lax.fori_loop(0, TPT // ZR, zero_body, None)
    plsc.subcore_barrier()

    bufs = (buf0, buf1, buf2, buf3)
    sems = (g0, g1, g2, g3)
    nb = len(bufs)
    for b in range(nb):
        pltpu.async_copy(y_hbm.at[ridx.at[b]], bufs[b], sems[b])

    def body(i, _):
        k = nb * i
        for b in range(nb):
            pltpu.make_async_copy(y_hbm.at[ridx.at[k + b]], bufs[b],
                                  sems[b]).wait()
            pltpu.sync_copy(bufs[b], acc.at[cidx.at[k + b]], add=True)

            @pl.when(k + b + nb < NCHUNK)
            def _g():
                pltpu.async_copy(y_hbm.at[ridx.at[k + b + nb]], bufs[b],
                                 sems[b])

        return _

    lax.fori_loop(0, NCHUNK // nb, body, None)
    tail = (NCHUNK // nb) * nb
    for b in range(NCHUNK - tail):
        pltpu.make_async_copy(y_hbm.at[ridx.at[tail + b]], bufs[b],
                              sems[b]).wait()
        pltpu.sync_copy(bufs[b], acc.at[cidx.at[tail + b]], add=True)

    plsc.subcore_barrier()

    def out_body(j, _):
        r0 = s * TPT + j * ZR
        pltpu.sync_copy(acc.at[pl.ds(r0, ZR)], zbuf)
        pltpu.sync_copy(zbuf, out_hbm.at[pl.ds(c * NP + r0, ZR)])
        return _

    lax.fori_loop(0, TPT // ZR, out_body, None)


TAILN = E2 - E


def _tc_edges_body(ei_ref, rowx_ref, colx_ref):
    ei = ei_ref[...]
    j = lax.broadcasted_iota(jnp.int32, (TAILN,), 0)
    rowx_ref[pl.ds(0, E)] = ei[0, :]
    rowx_ref[pl.ds(E, TAILN)] = jnp.where(j < N, j, j - N)
    colx_ref[pl.ds(0, E)] = ei[1, :]
    colx_ref[pl.ds(E, TAILN)] = jnp.where(j < N, j, N + (j - N) % (NP - N))


def _tc_edges(edge_index):
    return pl.pallas_call(
        _tc_edges_body,
        out_shape=[
            jax.ShapeDtypeStruct((E2,), jnp.int32),
            jax.ShapeDtypeStruct((E2,), jnp.int32),
        ],
    )(edge_index)


DGR = NP * 8 // 128


def _tc_dinv_body(deg_ref, p_ref, dinv_ref):
    v = deg_ref[...]
    dall = lax.rsqrt(v[:DGR] + v[DGR:])
    p = p_ref[...]
    sels = [jnp.dot(dall, p[128 * q:128 * (q + 1)],
                    preferred_element_type=jnp.float32,
                    precision=lax.Precision.HIGHEST)[:, None, :]
            for q in range(8)]
    out = jnp.concatenate(sels, axis=1).reshape(8 * DGR, 128)
    dinv_ref[...] = out[:NH]


def _tc_dinv(deg_parts, perms):
    return pl.pallas_call(
        _tc_dinv_body,
        out_shape=jax.ShapeDtypeStruct((NH, 2 * H), jnp.float32),
    )(deg_parts.reshape(2 * DGR, 128), perms)


def _dinv_perms():
    l64 = (jnp.arange(128) >= 64).astype(jnp.int32)
    rows = []
    for q in range(8):
        src = 16 * q + 8 * l64
        rows.append((jnp.arange(128)[:, None] == src[None, :])
                    .astype(jnp.float32))
    return jnp.concatenate(rows, axis=0)


def _tc_y0_body(x2_ref, w0_ref, dinv_ref, y_ref):
    xw = jnp.dot(x2_ref[...], w0_ref[...], preferred_element_type=jnp.float32)
    y_ref[...] = xw * dinv_ref[...]


def _tc_y0(x2, W0blk, dinv128):
    return pl.pallas_call(
        _tc_y0_body,
        out_shape=jax.ShapeDtypeStruct((NH, 2 * H), jnp.float32),
    )(x2, W0blk, dinv128)


def _bn_relu(o, g2, be2):
    mu = jnp.mean(o, axis=0, keepdims=True)
    sq = jnp.mean(o * o, axis=0, keepdims=True)
    mu64 = (mu[:, :H] + mu[:, H:]) * 0.5
    sq64 = (sq[:, :H] + sq[:, H:]) * 0.5
    var64 = sq64 - mu64 * mu64
    mu2 = jnp.concatenate([mu64, mu64], axis=1)
    rstd2 = jnp.concatenate([lax.rsqrt(var64 + 1e-5),
                             lax.rsqrt(var64 + 1e-5)], axis=1)
    return jnp.maximum((o - mu2) * rstd2 * g2 + be2, 0.0)


def _tc_post_body(sp_ref, dinv_ref, b_ref, g_ref, be_ref, wn_ref, yn_ref):
    dinv = dinv_ref[...]
    sp = sp_ref[...]
    o = dinv * (sp[:NH] + sp[SPR:SPR + NH]) + b_ref[...]
    h = _bn_relu(o, g_ref[...], be_ref[...])
    yn_ref[...] = jnp.dot(h, wn_ref[...],
                          preferred_element_type=jnp.float32) * dinv


def _tc_post(sp128, dinv128, b2, g2, be2, Wnblk):
    return pl.pallas_call(
        _tc_post_body,
        out_shape=jax.ShapeDtypeStruct((NH, 2 * H), jnp.float32),
    )(sp128, dinv128, b2, g2, be2, Wnblk)


def _tc_final_body(sp_ref, dinv_ref, b_ref, g_ref, be_ref,
                   wc1_ref, bc1_ref, wc2_ref, bc2_ref, out_ref):
    sp = sp_ref[...]
    o = dinv_ref[...] * (sp[:NH] + sp[SPR:SPR + NH]) + b_ref[...]
    h = _bn_relu(o, g_ref[...], be_ref[...])
    hc = jnp.maximum(
        jnp.dot(h, wc1_ref[...], preferred_element_type=jnp.float32)
        + bc1_ref[...], 0.0)
    out_ref[...] = (
        jnp.dot(hc, wc2_ref[...], preferred_element_type=jnp.float32)
        + bc2_ref[...])


def _tc_final(sp128, dinv128, b2, g2, be2, Wc1blk, bc1_2, Wc2blk, bc2_2):
    return pl.pallas_call(
        _tc_final_body,
        out_shape=jax.ShapeDtypeStruct((NH, 2 * C), jnp.float32),
    )(sp128, dinv128, b2, g2, be2, Wc1blk, bc1_2, Wc2blk, bc2_2)


def _blockdiag(W):
    fi, fo = W.shape
    Z = jnp.zeros((fi, fo), W.dtype)
    return jnp.concatenate(
        [jnp.concatenate([W, Z], axis=1), jnp.concatenate([Z, W], axis=1)],
        axis=0)


def _pair(v):
    return jnp.concatenate([v, v]).reshape(1, 2 * v.shape[0])


def kernel(x, edge_index, W0, b0, W1, b1, W2, b2, g0, be0, g1, be1, g2, be2,
           Wc1, bc1, Wc2, bc2):
    rowx, colx = _tc_edges(edge_index)
    row3 = rowx.reshape(NW, NCHUNK, CH)
    col3 = colx.reshape(NW, NCHUNK, CH)
    ones8 = jnp.ones((CH, 8), jnp.float32)
    zeros8 = jnp.zeros((ZR, 8), jnp.float32)
    zerosH = jnp.zeros((ZR, H), jnp.float32)

    x2 = x.reshape(NH, 2 * F_IN)

    deg_parts = _sc_degree(col3, ones8, zeros8)
    dinv128 = _tc_dinv(deg_parts, _dinv_perms())
    y0 = _tc_y0(x2, _blockdiag(W0), dinv128)

    s0 = _sc_aggregate(y0.reshape(2 * NH, H), row3, col3,
                       zerosH).reshape(NC * NP * H // (2 * H), 2 * H)
    y1 = _tc_post(s0, dinv128, _pair(b0), _pair(g0), _pair(be0),
                  _blockdiag(W1))
    s1 = _sc_aggregate(y1.reshape(2 * NH, H), row3, col3,
                       zerosH).reshape(NC * NP * H // (2 * H), 2 * H)
    y2 = _tc_post(s1, dinv128, _pair(b1), _pair(g1), _pair(be1),
                  _blockdiag(W2))
    s2 = _sc_aggregate(y2.reshape(2 * NH, H), row3, col3,
                       zerosH).reshape(NC * NP * H // (2 * H), 2 * H)
    out2 = _tc_final(s2, dinv128, _pair(b2), _pair(g2), _pair(be2),
                     _blockdiag(Wc1), _pair(bc1), _blockdiag(Wc2), _pair(bc2))
    return out2.reshape(N, C)


# speedup vs baseline: 1.0550x; 1.0053x over previous
import functools

import jax
import jax.numpy as jnp
from jax import lax
from jax.experimental import pallas as pl
from jax.experimental.pallas import tpu as pltpu
from jax.experimental.pallas import tpu_sc as plsc

N = 10000
E = 320000
F_IN = 128
H = 64
C = 10

NC = 2
NS = 16
NW = NC * NS
CH = 128
NCHUNK = 81
EPW = NCHUNK * CH
E2 = EPW * NW
PAD = E2 - E - N
NP = 10112
TPT = NP // NS
ZR = 158
NH = 5000
SPR = NP * H // (2 * H)

_MESH = plsc.VectorSubcoreMesh(core_axis_name="c", subcore_axis_name="s")
_SC_PARAMS = pltpu.CompilerParams(use_tc_tiling_on_sc=False,
                                  disable_bounds_checks=True)


@functools.partial(
    pl.kernel,
    out_type=jax.ShapeDtypeStruct((NC * NP, 8), jnp.float32),
    mesh=_MESH,
    compiler_params=_SC_PARAMS,
    scratch_types=[
        pltpu.VMEM((NCHUNK, CH), jnp.int32),
        pltpu.VMEM((CH, 8), jnp.float32),
        pltpu.VMEM((ZR, 8), jnp.float32),
        pltpu.VMEM_SHARED((NP, 8), jnp.float32),
    ],
)
def _sc_degree(col_hbm, ones_hbm, zeros_hbm, out_hbm, cidx, ones_v, zbuf, acc):
    c = lax.axis_index("c")
    s = lax.axis_index("s")
    wid = s * NC + c

    pltpu.sync_copy(col_hbm.at[wid], cidx)
    pltpu.sync_copy(ones_hbm, ones_v)
    pltpu.sync_copy(zeros_hbm, zbuf)

    def zero_body(j, _):
        pltpu.sync_copy(zbuf, acc.at[pl.ds(s * TPT + j * ZR, ZR)])
        return _

    lax.fori_loop(0, TPT // ZR, zero_body, None)
    plsc.subcore_barrier()

    def body(k, _):
        pltpu.sync_copy(ones_v, acc.at[cidx.at[k]], add=True)
        return _

    lax.fori_loop(0, NCHUNK, body, None)
    plsc.subcore_barrier()

    def out_body(j, _):
        r0 = s * TPT + j * ZR
        pltpu.sync_copy(acc.at[pl.ds(r0, ZR)], zbuf)
        pltpu.sync_copy(zbuf, out_hbm.at[pl.ds(c * NP + r0, ZR)])
        return _

    lax.fori_loop(0, TPT // ZR, out_body, None)


@functools.partial(
    pl.kernel,
    out_type=jax.ShapeDtypeStruct((NC * NP, H), jnp.float32),
    mesh=_MESH,
    compiler_params=_SC_PARAMS,
    scratch_types=[
        pltpu.VMEM((NCHUNK, CH), jnp.int32),
        pltpu.VMEM((NCHUNK, CH), jnp.int32),
        pltpu.VMEM((CH, H), jnp.float32),
        pltpu.VMEM((CH, H), jnp.float32),
        pltpu.VMEM((CH, H), jnp.float32),
        pltpu.VMEM((CH, H), jnp.float32),
        pltpu.VMEM((CH, H), jnp.float32),
        pltpu.VMEM((CH, H), jnp.float32),
        pltpu.VMEM((ZR, H), jnp.float32),
        pltpu.VMEM_SHARED((NP, H), jnp.float32),
        pltpu.SemaphoreType.DMA,
        pltpu.SemaphoreType.DMA,
        pltpu.SemaphoreType.DMA,
        pltpu.SemaphoreType.DMA,
        pltpu.SemaphoreType.DMA,
        pltpu.SemaphoreType.DMA,
    ],
)
def _sc_aggregate(y_hbm, row_hbm, col_hbm, zeros_hbm, out_hbm,
                  ridx, cidx, buf0, buf1, buf2, buf3, buf4, buf5, zbuf, acc,
                  g0, g1, g2, g3, g4, g5):
    c = lax.axis_index("c")
    s = lax.axis_index("s")
    wid = s * NC + c

    pltpu.sync_copy(row_hbm.at[wid], ridx)
    pltpu.sync_copy(col_hbm.at[wid], cidx)
    pltpu.sync_copy(zeros_hbm, zbuf)

    def zero_body(j, _):
        pltpu.sync_copy(zbuf, acc.at[pl.ds(s * TPT + j * ZR, ZR)])
        return _

    lax.fori_loop(0, TPT // ZR, zero_body, None)
    plsc.subcore_barrier()

    bufs = (buf0, buf1, buf2, buf3, buf4, buf5)
    sems = (g0, g1, g2, g3, g4, g5)
    nb = len(bufs)
    for b in range(nb):
        pltpu.async_copy(y_hbm.at[ridx.at[b]], bufs[b], sems[b])

    def body(i, _):
        k = nb * i
        for b in range(nb):
            pltpu.make_async_copy(y_hbm.at[ridx.at[k + b]], bufs[b],
                                  sems[b]).wait()
            pltpu.sync_copy(bufs[b], acc.at[cidx.at[k + b]], add=True)

            @pl.when(k + b + nb < NCHUNK)
            def _g():
                pltpu.async_copy(y_hbm.at[ridx.at[k + b + nb]], bufs[b],
                                 sems[b])

        return _

    lax.fori_loop(0, NCHUNK // nb, body, None)
    tail = (NCHUNK // nb) * nb
    for b in range(NCHUNK - tail):
        pltpu.make_async_copy(y_hbm.at[ridx.at[tail + b]], bufs[b],
                              sems[b]).wait()
        pltpu.sync_copy(bufs[b], acc.at[cidx.at[tail + b]], add=True)

    plsc.subcore_barrier()

    def out_body(j, _):
        r0 = s * TPT + j * ZR
        pltpu.sync_copy(acc.at[pl.ds(r0, ZR)], zbuf)
        pltpu.sync_copy(zbuf, out_hbm.at[pl.ds(c * NP + r0, ZR)])
        return _

    lax.fori_loop(0, TPT // ZR, out_body, None)


TAILN = E2 - E


def _tc_edges_body(ei_ref, rowx_ref, colx_ref):
    ei = ei_ref[...]
    j = lax.broadcasted_iota(jnp.int32, (TAILN,), 0)
    rowx_ref[pl.ds(0, E)] = ei[0, :]
    rowx_ref[pl.ds(E, TAILN)] = jnp.where(j < N, j, j - N)
    colx_ref[pl.ds(0, E)] = ei[1, :]
    colx_ref[pl.ds(E, TAILN)] = jnp.where(j < N, j, N + (j - N) % (NP - N))


def _tc_edges(edge_index):
    return pl.pallas_call(
        _tc_edges_body,
        out_shape=[
            jax.ShapeDtypeStruct((E2,), jnp.int32),
            jax.ShapeDtypeStruct((E2,), jnp.int32),
        ],
    )(edge_index)


DGR = NP * 8 // 128


def _tc_dinv_body(deg_ref, p_ref, dinv_ref):
    v = deg_ref[...]
    dall = lax.rsqrt(v[:DGR] + v[DGR:])
    p = p_ref[...]
    sels = [jnp.dot(dall, p[128 * q:128 * (q + 1)],
                    preferred_element_type=jnp.float32,
                    precision=lax.Precision.HIGHEST)[:, None, :]
            for q in range(8)]
    out = jnp.concatenate(sels, axis=1).reshape(8 * DGR, 128)
    dinv_ref[...] = out[:NH]


def _tc_dinv(deg_parts, perms):
    return pl.pallas_call(
        _tc_dinv_body,
        out_shape=jax.ShapeDtypeStruct((NH, 2 * H), jnp.float32),
    )(deg_parts.reshape(2 * DGR, 128), perms)


def _dinv_perms():
    l64 = (jnp.arange(128) >= 64).astype(jnp.int32)
    rows = []
    for q in range(8):
        src = 16 * q + 8 * l64
        rows.append((jnp.arange(128)[:, None] == src[None, :])
                    .astype(jnp.float32))
    return jnp.concatenate(rows, axis=0)


def _tc_y0_body(x2_ref, w0_ref, dinv_ref, y_ref):
    xw = jnp.dot(x2_ref[...], w0_ref[...], preferred_element_type=jnp.float32)
    y_ref[...] = xw * dinv_ref[...]


def _tc_y0(x2, W0blk, dinv128):
    return pl.pallas_call(
        _tc_y0_body,
        out_shape=jax.ShapeDtypeStruct((NH, 2 * H), jnp.float32),
    )(x2, W0blk, dinv128)


def _bn_relu(o, g2, be2):
    mu = jnp.mean(o, axis=0, keepdims=True)
    sq = jnp.mean(o * o, axis=0, keepdims=True)
    mu64 = (mu[:, :H] + mu[:, H:]) * 0.5
    sq64 = (sq[:, :H] + sq[:, H:]) * 0.5
    var64 = sq64 - mu64 * mu64
    mu2 = jnp.concatenate([mu64, mu64], axis=1)
    rstd2 = jnp.concatenate([lax.rsqrt(var64 + 1e-5),
                             lax.rsqrt(var64 + 1e-5)], axis=1)
    return jnp.maximum((o - mu2) * rstd2 * g2 + be2, 0.0)


def _tc_post_body(sp_ref, dinv_ref, b_ref, g_ref, be_ref, wn_ref, yn_ref):
    dinv = dinv_ref[...]
    sp = sp_ref[...]
    o = dinv * (sp[:NH] + sp[SPR:SPR + NH]) + b_ref[...]
    h = _bn_relu(o, g_ref[...], be_ref[...])
    yn_ref[...] = jnp.dot(h, wn_ref[...],
                          preferred_element_type=jnp.float32) * dinv


def _tc_post(sp128, dinv128, b2, g2, be2, Wnblk):
    return pl.pallas_call(
        _tc_post_body,
        out_shape=jax.ShapeDtypeStruct((NH, 2 * H), jnp.float32),
    )(sp128, dinv128, b2, g2, be2, Wnblk)


def _tc_final_body(sp_ref, dinv_ref, b_ref, g_ref, be_ref,
                   wc1_ref, bc1_ref, wc2_ref, bc2_ref, out_ref):
    sp = sp_ref[...]
    o = dinv_ref[...] * (sp[:NH] + sp[SPR:SPR + NH]) + b_ref[...]
    h = _bn_relu(o, g_ref[...], be_ref[...])
    hc = jnp.maximum(
        jnp.dot(h, wc1_ref[...], preferred_element_type=jnp.float32)
        + bc1_ref[...], 0.0)
    out_ref[...] = (
        jnp.dot(hc, wc2_ref[...], preferred_element_type=jnp.float32)
        + bc2_ref[...])


def _tc_final(sp128, dinv128, b2, g2, be2, Wc1blk, bc1_2, Wc2blk, bc2_2):
    return pl.pallas_call(
        _tc_final_body,
        out_shape=jax.ShapeDtypeStruct((NH, 2 * C), jnp.float32),
    )(sp128, dinv128, b2, g2, be2, Wc1blk, bc1_2, Wc2blk, bc2_2)


def _blockdiag(W):
    fi, fo = W.shape
    Z = jnp.zeros((fi, fo), W.dtype)
    return jnp.concatenate(
        [jnp.concatenate([W, Z], axis=1), jnp.concatenate([Z, W], axis=1)],
        axis=0)


def _pair(v):
    return jnp.concatenate([v, v]).reshape(1, 2 * v.shape[0])


def kernel(x, edge_index, W0, b0, W1, b1, W2, b2, g0, be0, g1, be1, g2, be2,
           Wc1, bc1, Wc2, bc2):
    rowx, colx = _tc_edges(edge_index)
    row3 = rowx.reshape(NW, NCHUNK, CH)
    col3 = colx.reshape(NW, NCHUNK, CH)
    ones8 = jnp.ones((CH, 8), jnp.float32)
    zeros8 = jnp.zeros((ZR, 8), jnp.float32)
    zerosH = jnp.zeros((ZR, H), jnp.float32)

    x2 = x.reshape(NH, 2 * F_IN)

    deg_parts = _sc_degree(col3, ones8, zeros8)
    dinv128 = _tc_dinv(deg_parts, _dinv_perms())
    y0 = _tc_y0(x2, _blockdiag(W0), dinv128)

    s0 = _sc_aggregate(y0.reshape(2 * NH, H), row3, col3,
                       zerosH).reshape(NC * NP * H // (2 * H), 2 * H)
    y1 = _tc_post(s0, dinv128, _pair(b0), _pair(g0), _pair(be0),
                  _blockdiag(W1))
    s1 = _sc_aggregate(y1.reshape(2 * NH, H), row3, col3,
                       zerosH).reshape(NC * NP * H // (2 * H), 2 * H)
    y2 = _tc_post(s1, dinv128, _pair(b1), _pair(g1), _pair(be1),
                  _blockdiag(W2))
    s2 = _sc_aggregate(y2.reshape(2 * NH, H), row3, col3,
                       zerosH).reshape(NC * NP * H // (2 * H), 2 * H)
    out2 = _tc_final(s2, dinv128, _pair(b2), _pair(g2), _pair(be2),
                     _blockdiag(Wc1), _pair(bc1), _blockdiag(Wc2), _pair(bc2))
    return out2.reshape(N, C)
